# Initial kernel scaffold; baseline (speedup 1.0000x reference)
import jax
import jax.numpy as jnp
from jax.experimental import pallas as pl


def kernel(inputs_values, unpool_mat):
    raise NotImplementedError("write your pallas kernel here")


# speedup vs baseline: 1.2244x; 1.2244x over previous
import jax
import jax.numpy as jnp
from jax import lax
from jax.experimental import pallas as pl
from jax.experimental.pallas import tpu as pltpu
from jax.experimental.pallas import tpu_sc as plsc

N_OUT = 4 * 384 * 384 * 96
N_UPD = 4 * 192 * 192 * 96
SHIFT = 20
BINW = 1 << SHIFT
NBINS = N_OUT // BINW
NB_PAD = 64
NCORES = 2
NSUB = 16
NW = NCORES * NSUB
CHUNK = N_UPD // NW
CHA = 8192
CHB = 8192
CHC = 8192
NSTG = N_UPD + NBINS * 8 + CHC
TRASH = 16 * NSUB


def _wid():
    return lax.axis_index("s") * NCORES + lax.axis_index("c")


def _hist_body(idx_hbm, counts_hbm, idx_v, hist_v, cnt_v):
    wid = _wid()
    base = wid * CHUNK
    lanebase = lax.iota(jnp.int32, 16) * NB_PAD
    ones = jnp.ones((16,), jnp.int32)

    def _z(i, _):
        hist_v[pl.ds(i * 16, 16)] = jnp.zeros((16,), jnp.int32)
        return 0
    lax.fori_loop(0, (16 * NB_PAD) // 16, _z, 0)

    def _win(w, _):
        pltpu.sync_copy(idx_hbm.at[pl.ds(base + w * CHA, CHA)], idx_v)

        def _vreg(j, _):
            iv = idx_v[pl.ds(j * 16, 16)]
            pos = lax.shift_right_logical(iv, SHIFT) + lanebase
            plsc.addupdate_scatter(hist_v, [pos], ones)
            return 0
        lax.fori_loop(0, CHA // 16, _vreg, 0)
        return 0
    lax.fori_loop(0, CHUNK // CHA, _win, 0)

    for g in range(NB_PAD // 16):
        acc = jnp.zeros((16,), jnp.int32)
        for l in range(16):
            acc = acc + hist_v[pl.ds(l * NB_PAD + g * 16, 16)]
        cnt_v[pl.ds(g * 16, 16)] = acc
    pltpu.sync_copy(cnt_v, counts_hbm.at[wid])


def _route_body(idx_hbm, val_hbm, counts_hbm,
                stgi_hbm, stgv_hbm, meta_hbm,
                idx_v, val_v, rel_v, dst_v, ctr_v, counts_v, meta_v):
    wid = _wid()
    base = wid * CHUNK

    pltpu.sync_copy(counts_hbm, counts_v)

    carry = jnp.int32(0)
    for g in range(NB_PAD // 16):
        tot = jnp.zeros((16,), jnp.int32)
        mine = jnp.zeros((16,), jnp.int32)
        for t in range(NW):
            row = counts_v[t, pl.ds(g * 16, 16)]
            tot = tot + row
            mine = mine + jnp.where(jnp.int32(t) < wid, row, 0)
        ptot = (tot + 7) & ~jnp.int32(7)
        excl = plsc.cumsum(ptot) - ptot
        off = excl + carry
        ctr_v[pl.ds(g * 16, 16)] = off + mine
        meta_v[pl.ds(g * 16, 16)] = off
        meta_v[pl.ds(NB_PAD + g * 16, 16)] = tot
        carry = carry + jnp.sum(ptot)

    @pl.when(wid == 0)
    def _():
        pltpu.sync_copy(meta_v, meta_hbm)

    mask20 = jnp.int32(BINW - 1)

    def _win(w, _):
        wb = base + w * CHB
        pltpu.sync_copy(idx_hbm.at[pl.ds(wb, CHB)], idx_v)
        pltpu.sync_copy(val_hbm.at[pl.ds(wb, CHB)], val_v)

        def _vreg(j, _):
            iv = idx_v[pl.ds(j * 16, 16)]
            b = lax.shift_right_logical(iv, SHIFT)
            cnt, last = plsc.scan_count(b)
            bse = plsc.load_gather(ctr_v, [b])
            dst_v[pl.ds(j * 16, 16)] = bse + cnt - 1
            rel_v[pl.ds(j * 16, 16)] = iv & mask20
            plsc.addupdate_scatter(ctr_v, [b], cnt, mask=last)
            return 0
        lax.fori_loop(0, CHB // 16, _vreg, 0)

        pltpu.sync_copy(rel_v, stgi_hbm.at[dst_v])
        pltpu.sync_copy(val_v, stgv_hbm.at[dst_v])
        return 0
    lax.fori_loop(0, CHUNK // CHB, _win, 0)


def _acc_body(stgi_hbm, stgv_hbm, meta_hbm, zeros_hbm, out_hbm,
              idx_v, val_v, meta_v, meta_sm, acc, meta_sp):
    cid = lax.axis_index("c")
    sid = lax.axis_index("s")
    lanes = lax.iota(jnp.int32, 16)
    trash_vec = jnp.int32(BINW) + sid * 16 + lanes
    zslice = BINW // NSUB

    pltpu.sync_copy(meta_hbm, meta_v)

    @pl.when(sid == 0)
    def _():
        pltpu.sync_copy(meta_v, meta_sp)
    plsc.subcore_barrier()
    pltpu.sync_copy(meta_sp, meta_sm)

    def _round(r, _):
        b = NCORES * r + cid
        a0 = meta_sm[b]
        ln = meta_sm[NB_PAD + b]

        pltpu.sync_copy(zeros_hbm.at[pl.ds(sid * zslice, zslice)],
                        acc.at[pl.ds(sid * zslice, zslice)])
        plsc.subcore_barrier()

        q = lax.shift_right_logical(ln, 4)
        st = a0 + sid * q
        en = jnp.where(sid == NSUB - 1, a0 + ln, st + q)
        al = pl.multiple_of(st & ~jnp.int32(7), 8)
        nwin = lax.shift_right_logical(en - al + CHC - 1, 13)

        def _win(w, _):
            wb = al + w * CHC
            pltpu.sync_copy(stgi_hbm.at[pl.ds(wb, CHC)], idx_v)
            pltpu.sync_copy(stgv_hbm.at[pl.ds(wb, CHC)], val_v)

            def _vreg(j, _):
                pos = wb + j * 16 + lanes
                iv = idx_v[pl.ds(j * 16, 16)]
                ok = (pos >= st) & (pos < en)
                idx_v[pl.ds(j * 16, 16)] = jnp.where(ok, iv, trash_vec)
                return 0
            lax.fori_loop(0, CHC // 16, _vreg, 0)

            pltpu.sync_copy(val_v, acc.at[idx_v], add=True)
            return 0
        lax.fori_loop(0, nwin, _win, 0)
        plsc.subcore_barrier()

        pltpu.sync_copy(acc.at[pl.ds(sid * zslice, zslice)],
                        out_hbm.at[pl.ds(b * BINW + sid * zslice, zslice)])
        plsc.subcore_barrier()
        return 0
    lax.fori_loop(0, NBINS // NCORES, _round, 0)


@jax.jit
def kernel(inputs_values, unpool_mat):
    vals = inputs_values.reshape(-1)
    idx = unpool_mat.reshape(-1)
    mesh = plsc.VectorSubcoreMesh(core_axis_name="c", subcore_axis_name="s")
    params = pltpu.CompilerParams(needs_layout_passes=False)

    counts = pl.kernel(
        _hist_body,
        out_type=jax.ShapeDtypeStruct((NW, NB_PAD), jnp.int32),
        mesh=mesh,
        scratch_types=[
            pltpu.VMEM((CHA,), jnp.int32),
            pltpu.VMEM((16 * NB_PAD,), jnp.int32),
            pltpu.VMEM((NB_PAD,), jnp.int32),
        ],
        compiler_params=params,
    )(idx)

    stgi, stgv, meta = pl.kernel(
        _route_body,
        out_type=(jax.ShapeDtypeStruct((NSTG,), jnp.int32),
                  jax.ShapeDtypeStruct((NSTG,), jnp.float32),
                  jax.ShapeDtypeStruct((2 * NB_PAD,), jnp.int32)),
        mesh=mesh,
        scratch_types=[
            pltpu.VMEM((CHB,), jnp.int32),
            pltpu.VMEM((CHB,), jnp.float32),
            pltpu.VMEM((CHB,), jnp.int32),
            pltpu.VMEM((CHB,), jnp.int32),
            pltpu.VMEM((NB_PAD,), jnp.int32),
            pltpu.VMEM((NW, NB_PAD), jnp.int32),
            pltpu.VMEM((2 * NB_PAD,), jnp.int32),
        ],
        compiler_params=params,
    )(idx, vals, counts)

    zeros = jnp.zeros((BINW,), jnp.float32)
    out = pl.kernel(
        _acc_body,
        out_type=jax.ShapeDtypeStruct((N_OUT,), jnp.float32),
        mesh=mesh,
        scratch_types=[
            pltpu.VMEM((CHC,), jnp.int32),
            pltpu.VMEM((CHC,), jnp.float32),
            pltpu.VMEM((2 * NB_PAD,), jnp.int32),
            pltpu.SMEM((2 * NB_PAD,), jnp.int32),
            pltpu.VMEM_SHARED((BINW + TRASH,), jnp.float32),
            pltpu.VMEM_SHARED((2 * NB_PAD,), jnp.int32),
        ],
        compiler_params=params,
    )(stgi, stgv, meta, zeros)

    return out.reshape(inputs_values.shape[0], 384, 384, 96)


# speedup vs baseline: 7.5742x; 6.1861x over previous
import jax
import jax.numpy as jnp
from jax import lax
from jax.experimental import pallas as pl
from jax.experimental.pallas import tpu as pltpu
from jax.experimental.pallas import tpu_sc as plsc

N_OUT = 4 * 384 * 384 * 96
N_UPD = 4 * 192 * 192 * 96
SHIFT = 20
BINW = 1 << SHIFT
NBINS = N_OUT // BINW
NB_PAD = 64
NCORES = 2
NSUB = 16
NW = NCORES * NSUB
CHUNK = N_UPD // NW
WSZ = 16384
WPT = CHUNK // WSZ
NSTRIP = NW * WPT
RW = 2048
NSTG = N_UPD + RW
TRASH = 16 * NSUB
SPT = NSTRIP // NSUB


def _route_body(idx_hbm, val_hbm, stgi_hbm, stgv_hbm, pref_hbm,
                idx_v, val_v, sidx_v, sval_v, whist_v, wctr_v, pref_v,
                dstp_v):
    wid = lax.axis_index("s") * NCORES + lax.axis_index("c")
    base = wid * CHUNK
    lanes = lax.iota(jnp.int32, 16)
    lanebase = lanes * NB_PAD
    ones = jnp.ones((16,), jnp.int32)
    mask20 = jnp.int32(BINW - 1)

    def _win(w, _):
        col = wid * WPT + w
        pltpu.sync_copy(idx_hbm.at[pl.ds(base + w * WSZ, WSZ)], idx_v)
        pltpu.sync_copy(val_hbm.at[pl.ds(base + w * WSZ, WSZ)], val_v)

        def _z(i, _):
            whist_v[pl.ds(i * 16, 16)] = jnp.zeros((16,), jnp.int32)
            return 0
        lax.fori_loop(0, (16 * NB_PAD) // 16, _z, 0)

        def _h(j, _):
            iv = idx_v[pl.ds(j * 16, 16)]
            pos = lax.shift_right_logical(iv, SHIFT) + lanebase
            plsc.addupdate_scatter(whist_v, [pos], ones)
            return 0
        lax.fori_loop(0, WSZ // 16, _h, 0)

        carry = jnp.int32(0)
        for g in range(NB_PAD // 16):
            tot = jnp.zeros((16,), jnp.int32)
            for l in range(16):
                tot = tot + whist_v[pl.ds(l * NB_PAD + g * 16, 16)]
            excl = plsc.cumsum(tot) - tot + carry
            wctr_v[pl.ds(g * 16, 16)] = excl
            pref_v[pl.ds(g * 16, 16)] = excl
            dstp_v[pl.ds(g * 16, 16)] = (g * 16 + lanes) * NSTRIP + col
            carry = carry + jnp.sum(tot)

        pltpu.sync_copy(pref_v, pref_hbm.at[dstp_v])

        def _r(j, _):
            iv = idx_v[pl.ds(j * 16, 16)]
            vv = val_v[pl.ds(j * 16, 16)]
            b = lax.shift_right_logical(iv, SHIFT)
            cnt, last = plsc.scan_count(b)
            bse = plsc.load_gather(wctr_v, [b])
            dst = bse + cnt - 1
            plsc.store_scatter(sidx_v, [dst], iv & mask20)
            plsc.store_scatter(sval_v, [dst], vv)
            plsc.addupdate_scatter(wctr_v, [b], cnt, mask=last)
            return 0
        lax.fori_loop(0, WSZ // 16, _r, 0)

        pltpu.sync_copy(sidx_v, stgi_hbm.at[pl.ds(col * WSZ, WSZ)])
        pltpu.sync_copy(sval_v, stgv_hbm.at[pl.ds(col * WSZ, WSZ)])
        return 0
    lax.fori_loop(0, WPT, _win, 0)


def _acc_body(stgi_hbm, stgv_hbm, pref_hbm, zeros_hbm, out_hbm,
              ridx_v, rval_v, prow_v, pref_sm, acc, pref_sp):
    cid = lax.axis_index("c")
    sid = lax.axis_index("s")
    lanes = lax.iota(jnp.int32, 16)
    trash_vec = jnp.int32(BINW) + sid * 16 + lanes
    zslice = BINW // NSUB

    def _round(r, _):
        b = NCORES * r + cid

        pltpu.sync_copy(zeros_hbm.at[pl.ds(sid * zslice, zslice)],
                        acc.at[pl.ds(sid * zslice, zslice)])

        pltpu.sync_copy(pref_hbm.at[pl.ds(b * NSTRIP, NSTRIP)],
                        prow_v.at[pl.ds(0, NSTRIP)])
        pltpu.sync_copy(pref_hbm.at[pl.ds((b + 1) * NSTRIP, NSTRIP)],
                        prow_v.at[pl.ds(NSTRIP, NSTRIP)])
        al_s = pl.multiple_of((sid * SPT) & ~jnp.int32(7), 8)
        pltpu.sync_copy(prow_v.at[pl.ds(al_s, 64)],
                        pref_sp.at[pl.ds(sid * 128, 64)])
        pltpu.sync_copy(prow_v.at[pl.ds(NSTRIP + al_s, 64)],
                        pref_sp.at[pl.ds(sid * 128 + 64, 64)])
        pltpu.sync_copy(pref_sp.at[pl.ds(sid * 128, 128)], pref_sm)
        plsc.subcore_barrier()

        def _run(k, _):
            s_k = sid * SPT + k
            st = s_k * WSZ + pref_sm[s_k - al_s]
            en = s_k * WSZ + pref_sm[64 + s_k - al_s]
            al = pl.multiple_of(st & ~jnp.int32(7), 8)
            nwin = jnp.where(
                en > st,
                lax.shift_right_logical(en - al + RW - 1, 11), 0)

            def _w(w, _):
                wb = al + w * RW
                pltpu.sync_copy(stgi_hbm.at[pl.ds(wb, RW)], ridx_v)
                pltpu.sync_copy(stgv_hbm.at[pl.ds(wb, RW)], rval_v)

                def _v(j, _):
                    pos = wb + j * 16 + lanes
                    iv = ridx_v[pl.ds(j * 16, 16)]
                    ok = (pos >= st) & (pos < en)
                    ridx_v[pl.ds(j * 16, 16)] = jnp.where(ok, iv, trash_vec)
                    return 0
                lax.fori_loop(0, RW // 16, _v, 0)

                pltpu.sync_copy(rval_v, acc.at[ridx_v], add=True)
                return 0
            lax.fori_loop(0, nwin, _w, 0)
            return 0
        lax.fori_loop(0, SPT, _run, 0)
        plsc.subcore_barrier()

        pltpu.sync_copy(acc.at[pl.ds(sid * zslice, zslice)],
                        out_hbm.at[pl.ds(b * BINW + sid * zslice, zslice)])
        plsc.subcore_barrier()
        return 0
    lax.fori_loop(0, NBINS // NCORES, _round, 0)


@jax.jit
def kernel(inputs_values, unpool_mat):
    vals = inputs_values.reshape(-1)
    idx = unpool_mat.reshape(-1)
    mesh = plsc.VectorSubcoreMesh(core_axis_name="c", subcore_axis_name="s")
    params = pltpu.CompilerParams(needs_layout_passes=False)

    stgi, stgv, pref = pl.kernel(
        _route_body,
        out_type=(jax.ShapeDtypeStruct((NSTG,), jnp.int32),
                  jax.ShapeDtypeStruct((NSTG,), jnp.float32),
                  jax.ShapeDtypeStruct(((NB_PAD + 1) * NSTRIP,), jnp.int32)),
        mesh=mesh,
        scratch_types=[
            pltpu.VMEM((WSZ,), jnp.int32),
            pltpu.VMEM((WSZ,), jnp.float32),
            pltpu.VMEM((WSZ,), jnp.int32),
            pltpu.VMEM((WSZ,), jnp.float32),
            pltpu.VMEM((16 * NB_PAD,), jnp.int32),
            pltpu.VMEM((NB_PAD,), jnp.int32),
            pltpu.VMEM((NB_PAD,), jnp.int32),
            pltpu.VMEM((NB_PAD,), jnp.int32),
        ],
        compiler_params=params,
    )(idx, vals)

    zeros = jnp.zeros((BINW,), jnp.float32)
    out = pl.kernel(
        _acc_body,
        out_type=jax.ShapeDtypeStruct((N_OUT,), jnp.float32),
        mesh=mesh,
        scratch_types=[
            pltpu.VMEM((RW,), jnp.int32),
            pltpu.VMEM((RW,), jnp.float32),
            pltpu.VMEM((2 * NSTRIP,), jnp.int32),
            pltpu.SMEM((128,), jnp.int32),
            pltpu.VMEM_SHARED((BINW + TRASH,), jnp.float32),
            pltpu.VMEM_SHARED((NSUB * 128,), jnp.int32),
        ],
        compiler_params=params,
    )(stgi, stgv, pref, zeros)

    return out.reshape(inputs_values.shape[0], 384, 384, 96)


# speedup vs baseline: 11.4884x; 1.5168x over previous
import jax
import jax.numpy as jnp
from jax import lax
from jax.experimental import pallas as pl
from jax.experimental.pallas import tpu as pltpu
from jax.experimental.pallas import tpu_sc as plsc

N_OUT = 4 * 384 * 384 * 96
N_UPD = 4 * 192 * 192 * 96
SHIFT = 20
BINW = 1 << SHIFT
NBINS = N_OUT // BINW
NB_PAD = 64
NCORES = 2
NSUB = 16
NW = NCORES * NSUB
CHUNK = N_UPD // NW
WSZ = 36864
WPT = CHUNK // WSZ
NSTRIP = NW * WPT
RW = 2048
NSTG = N_UPD + RW
TRASH = 16 * NSUB
SPT = NSTRIP // NSUB
ZB = 8192


def _route_body(idx_hbm, val_hbm, stgi_hbm, stgv_hbm, pref_hbm,
                a_v, dstb_v, c_v, whist_v, wctr_v, pref_v, dstp_v):
    wid = lax.axis_index("s") * NCORES + lax.axis_index("c")
    base = wid * CHUNK
    lanes = lax.iota(jnp.int32, 16)
    lanebase = lanes * NB_PAD
    ones = jnp.ones((16,), jnp.int32)
    mask20 = jnp.int32(BINW - 1)

    def _win(w, _):
        col = wid * WPT + w
        pltpu.sync_copy(idx_hbm.at[pl.ds(base + w * WSZ, WSZ)], a_v)

        def _z(i, _):
            whist_v[pl.ds(i * 16, 16)] = jnp.zeros((16,), jnp.int32)
            return 0
        lax.fori_loop(0, (16 * NB_PAD) // 16, _z, 0)

        def _h(j, _):
            iv = a_v[pl.ds(j * 16, 16)]
            pos = lax.shift_right_logical(iv, SHIFT) + lanebase
            plsc.addupdate_scatter(whist_v, [pos], ones)
            return 0
        lax.fori_loop(0, WSZ // 16, _h, 0)

        carry = jnp.int32(0)
        for g in range(NB_PAD // 16):
            tot = jnp.zeros((16,), jnp.int32)
            for l in range(16):
                tot = tot + whist_v[pl.ds(l * NB_PAD + g * 16, 16)]
            excl = plsc.cumsum(tot) - tot + carry
            wctr_v[pl.ds(g * 16, 16)] = excl
            pref_v[pl.ds(g * 16, 16)] = excl
            dstp_v[pl.ds(g * 16, 16)] = (g * 16 + lanes) * NSTRIP + col
            carry = carry + jnp.sum(tot)

        pltpu.sync_copy(pref_v, pref_hbm.at[dstp_v])

        def _r(j, _):
            iv = a_v[pl.ds(j * 16, 16)]
            b = lax.shift_right_logical(iv, SHIFT)
            cnt, last = plsc.scan_count(b)
            bse = plsc.load_gather(wctr_v, [b])
            dst = bse + cnt - 1
            dstb_v[pl.ds(j * 16, 16)] = dst
            plsc.store_scatter(c_v, [dst], iv & mask20)
            plsc.addupdate_scatter(wctr_v, [b], cnt, mask=last)
            return 0
        lax.fori_loop(0, WSZ // 16, _r, 0)
        pltpu.sync_copy(c_v, stgi_hbm.at[pl.ds(col * WSZ, WSZ)])

        pltpu.sync_copy(val_hbm.at[pl.ds(base + w * WSZ, WSZ)], a_v)

        def _r2(j, _):
            vv = a_v[pl.ds(j * 16, 16)]
            dst = dstb_v[pl.ds(j * 16, 16)]
            plsc.store_scatter(c_v, [dst], vv)
            return 0
        lax.fori_loop(0, WSZ // 16, _r2, 0)
        pltpu.sync_copy(c_v, stgv_hbm.at[pl.ds(col * WSZ, WSZ)])
        return 0
    lax.fori_loop(0, WPT, _win, 0)


def _acc_body(stgi_hbm, stgv_hbm, pref_hbm, out_hbm,
              ridx_v, rval_v, prow_v, zero_v, pref_sm, acc, pref_sp):
    cid = lax.axis_index("c")
    sid = lax.axis_index("s")
    lanes = lax.iota(jnp.int32, 16)
    trash_vec = jnp.int32(BINW) + sid * 16 + lanes
    zslice = BINW // NSUB

    def _zz(i, _):
        zero_v[pl.ds(i * 16, 16)] = jnp.zeros((16,), jnp.float32)
        return 0
    lax.fori_loop(0, ZB // 16, _zz, 0)

    def _round(r, _):
        b = NCORES * r + cid

        for z in range(zslice // ZB):
            pltpu.sync_copy(zero_v,
                            acc.at[pl.ds(sid * zslice + z * ZB, ZB)])

        pltpu.sync_copy(pref_hbm.at[pl.ds(b * NSTRIP, NSTRIP)],
                        prow_v.at[pl.ds(0, NSTRIP)])
        pltpu.sync_copy(pref_hbm.at[pl.ds((b + 1) * NSTRIP, NSTRIP)],
                        prow_v.at[pl.ds(NSTRIP, NSTRIP)])
        al_s = pl.multiple_of((sid * SPT) & ~jnp.int32(7), 8)
        pltpu.sync_copy(prow_v.at[pl.ds(al_s, 64)],
                        pref_sp.at[pl.ds(sid * 128, 64)])
        pltpu.sync_copy(prow_v.at[pl.ds(NSTRIP + al_s, 64)],
                        pref_sp.at[pl.ds(sid * 128 + 64, 64)])
        pltpu.sync_copy(pref_sp.at[pl.ds(sid * 128, 128)], pref_sm)
        plsc.subcore_barrier()

        def _run(k, _):
            s_k = sid * SPT + k
            st = s_k * WSZ + pref_sm[s_k - al_s]
            en = s_k * WSZ + pref_sm[64 + s_k - al_s]
            al = pl.multiple_of(st & ~jnp.int32(7), 8)
            nwin = jnp.where(
                en > st,
                lax.shift_right_logical(en - al + RW - 1, 11), 0)

            def _w(w, _):
                wb = al + w * RW
                pltpu.sync_copy(stgi_hbm.at[pl.ds(wb, RW)], ridx_v)
                pltpu.sync_copy(stgv_hbm.at[pl.ds(wb, RW)], rval_v)

                def _v(j, _):
                    pos = wb + j * 16 + lanes
                    iv = ridx_v[pl.ds(j * 16, 16)]
                    ok = (pos >= st) & (pos < en)
                    ridx_v[pl.ds(j * 16, 16)] = jnp.where(ok, iv, trash_vec)
                    return 0
                lax.fori_loop(0, RW // 16, _v, 0)

                pltpu.sync_copy(rval_v, acc.at[ridx_v], add=True)
                return 0
            lax.fori_loop(0, nwin, _w, 0)
            return 0
        lax.fori_loop(0, SPT, _run, 0)
        plsc.subcore_barrier()

        pltpu.sync_copy(acc.at[pl.ds(sid * zslice, zslice)],
                        out_hbm.at[pl.ds(b * BINW + sid * zslice, zslice)])
        plsc.subcore_barrier()
        return 0
    lax.fori_loop(0, NBINS // NCORES, _round, 0)


@jax.jit
def kernel(inputs_values, unpool_mat):
    vals_bits = lax.bitcast_convert_type(inputs_values.reshape(-1),
                                         jnp.int32)
    idx = unpool_mat.reshape(-1)
    mesh = plsc.VectorSubcoreMesh(core_axis_name="c", subcore_axis_name="s")
    params = pltpu.CompilerParams(needs_layout_passes=False)

    stgi, stgv, pref = pl.kernel(
        _route_body,
        out_type=(jax.ShapeDtypeStruct((NSTG,), jnp.int32),
                  jax.ShapeDtypeStruct((NSTG,), jnp.int32),
                  jax.ShapeDtypeStruct(((NB_PAD + 1) * NSTRIP,), jnp.int32)),
        mesh=mesh,
        scratch_types=[
            pltpu.VMEM((WSZ,), jnp.int32),
            pltpu.VMEM((WSZ,), jnp.int32),
            pltpu.VMEM((WSZ,), jnp.int32),
            pltpu.VMEM((16 * NB_PAD,), jnp.int32),
            pltpu.VMEM((NB_PAD,), jnp.int32),
            pltpu.VMEM((NB_PAD,), jnp.int32),
            pltpu.VMEM((NB_PAD,), jnp.int32),
        ],
        compiler_params=params,
    )(idx, vals_bits)

    stgv_f = lax.bitcast_convert_type(stgv, jnp.float32)
    out = pl.kernel(
        _acc_body,
        out_type=jax.ShapeDtypeStruct((N_OUT,), jnp.float32),
        mesh=mesh,
        scratch_types=[
            pltpu.VMEM((RW,), jnp.int32),
            pltpu.VMEM((RW,), jnp.float32),
            pltpu.VMEM((2 * NSTRIP,), jnp.int32),
            pltpu.VMEM((ZB,), jnp.float32),
            pltpu.SMEM((128,), jnp.int32),
            pltpu.VMEM_SHARED((BINW + TRASH,), jnp.float32),
            pltpu.VMEM_SHARED((NSUB * 128,), jnp.int32),
        ],
        compiler_params=params,
    )(stgi, stgv_f, pref)

    return out.reshape(inputs_values.shape[0], 384, 384, 96)


# speedup vs baseline: 13.1707x; 1.1464x over previous
import jax
import jax.numpy as jnp
from jax import lax
from jax.experimental import pallas as pl
from jax.experimental.pallas import tpu as pltpu
from jax.experimental.pallas import tpu_sc as plsc

N_OUT = 4 * 384 * 384 * 96
N_UPD = 4 * 192 * 192 * 96
SHIFT = 20
BINW = 1 << SHIFT
NBINS = N_OUT // BINW
NB_PAD = 64
NCORES = 2
NSUB = 16
NW = NCORES * NSUB
CHUNK = N_UPD // NW
WSZ = 36864
WPT = CHUNK // WSZ
NSTRIP = NW * WPT
RW = 1024
NSTG = N_UPD + RW
TRASH = 16 * NSUB
SPT = NSTRIP // NSUB
ZB = 8192


def _route_body(idx_hbm, val_hbm, stgi_hbm, stgv_hbm, pref_hbm,
                a_v, dstb_v, c_v, whist_v, wctr_v, pref_v, dstp_v):
    wid = lax.axis_index("s") * NCORES + lax.axis_index("c")
    base = wid * CHUNK
    lanes = lax.iota(jnp.int32, 16)
    lanebase = lanes * NB_PAD
    ones = jnp.ones((16,), jnp.int32)
    mask20 = jnp.int32(BINW - 1)

    def _win(w, _):
        col = wid * WPT + w
        pltpu.sync_copy(idx_hbm.at[pl.ds(base + w * WSZ, WSZ)], a_v)

        def _z(i, _):
            whist_v[pl.ds(i * 16, 16)] = jnp.zeros((16,), jnp.int32)
            return 0
        lax.fori_loop(0, (16 * NB_PAD) // 16, _z, 0)

        def _h(j, _):
            iv = a_v[pl.ds(j * 16, 16)]
            pos = lax.shift_right_logical(iv, SHIFT) + lanebase
            plsc.addupdate_scatter(whist_v, [pos], ones)
            return 0
        lax.fori_loop(0, WSZ // 16, _h, 0)

        carry = jnp.int32(0)
        for g in range(NB_PAD // 16):
            tot = jnp.zeros((16,), jnp.int32)
            for l in range(16):
                tot = tot + whist_v[pl.ds(l * NB_PAD + g * 16, 16)]
            excl = plsc.cumsum(tot) - tot + carry
            wctr_v[pl.ds(g * 16, 16)] = excl
            pref_v[pl.ds(g * 16, 16)] = excl
            dstp_v[pl.ds(g * 16, 16)] = (g * 16 + lanes) * NSTRIP + col
            carry = carry + jnp.sum(tot)

        pltpu.sync_copy(pref_v, pref_hbm.at[dstp_v])

        def _r(j, _):
            iv = a_v[pl.ds(j * 16, 16)]
            b = lax.shift_right_logical(iv, SHIFT)
            cnt, last = plsc.scan_count(b)
            bse = plsc.load_gather(wctr_v, [b])
            dst = bse + cnt - 1
            dstb_v[pl.ds(j * 16, 16)] = dst
            plsc.store_scatter(c_v, [dst], iv & mask20)
            plsc.addupdate_scatter(wctr_v, [b], cnt, mask=last)
            return 0
        lax.fori_loop(0, WSZ // 16, _r, 0)
        pltpu.sync_copy(c_v, stgi_hbm.at[pl.ds(col * WSZ, WSZ)])

        pltpu.sync_copy(val_hbm.at[pl.ds(base + w * WSZ, WSZ)], a_v)

        def _r2(j, _):
            vv = a_v[pl.ds(j * 16, 16)]
            dst = dstb_v[pl.ds(j * 16, 16)]
            plsc.store_scatter(c_v, [dst], vv)
            return 0
        lax.fori_loop(0, WSZ // 16, _r2, 0)
        pltpu.sync_copy(c_v, stgv_hbm.at[pl.ds(col * WSZ, WSZ)])
        return 0
    lax.fori_loop(0, WPT, _win, 0)


def _acc_body(stgi_hbm, stgv_hbm, pref_hbm, out_hbm,
              ridx_v, rval_v, prow_v, zero_v, pref_sm, acc, pref_sp):
    cid = lax.axis_index("c")
    sid = lax.axis_index("s")
    lanes = lax.iota(jnp.int32, 16)
    trash_vec = jnp.int32(BINW) + sid * 16 + lanes
    zslice = BINW // NSUB

    def _zz(i, _):
        zero_v[pl.ds(i * 16, 16)] = jnp.zeros((16,), jnp.float32)
        return 0
    lax.fori_loop(0, ZB // 16, _zz, 0)

    def _round(r, _):
        b = NCORES * r + cid

        for z in range(zslice // ZB):
            pltpu.sync_copy(zero_v,
                            acc.at[pl.ds(sid * zslice + z * ZB, ZB)])

        pltpu.sync_copy(pref_hbm.at[pl.ds(b * NSTRIP, NSTRIP)],
                        prow_v.at[pl.ds(0, NSTRIP)])
        pltpu.sync_copy(pref_hbm.at[pl.ds((b + 1) * NSTRIP, NSTRIP)],
                        prow_v.at[pl.ds(NSTRIP, NSTRIP)])
        al_s = pl.multiple_of((sid * SPT) & ~jnp.int32(7), 8)
        pltpu.sync_copy(prow_v.at[pl.ds(al_s, 64)],
                        pref_sp.at[pl.ds(sid * 128, 64)])
        pltpu.sync_copy(prow_v.at[pl.ds(NSTRIP + al_s, 64)],
                        pref_sp.at[pl.ds(sid * 128 + 64, 64)])
        pltpu.sync_copy(pref_sp.at[pl.ds(sid * 128, 128)], pref_sm)
        plsc.subcore_barrier()

        def _run(k, _):
            s_k = sid * SPT + k
            st = s_k * WSZ + pref_sm[s_k - al_s]
            en = s_k * WSZ + pref_sm[64 + s_k - al_s]
            al = pl.multiple_of(st & ~jnp.int32(7), 8)
            nwin = jnp.where(
                en > st,
                lax.shift_right_logical(en - al + RW - 1, 10), 0)

            def _w(w, _):
                wb = al + w * RW
                pltpu.sync_copy(stgi_hbm.at[pl.ds(wb, RW)], ridx_v)
                pltpu.sync_copy(stgv_hbm.at[pl.ds(wb, RW)], rval_v)

                def _v(j, _):
                    pos = wb + j * 16 + lanes
                    iv = ridx_v[pl.ds(j * 16, 16)]
                    ok = (pos >= st) & (pos < en)
                    ridx_v[pl.ds(j * 16, 16)] = jnp.where(ok, iv, trash_vec)
                    return 0
                lax.fori_loop(0, RW // 16, _v, 0)

                pltpu.sync_copy(rval_v, acc.at[ridx_v], add=True)
                return 0
            lax.fori_loop(0, nwin, _w, 0)
            return 0
        lax.fori_loop(0, SPT, _run, 0)
        plsc.subcore_barrier()

        pltpu.sync_copy(acc.at[pl.ds(sid * zslice, zslice)],
                        out_hbm.at[pl.ds(b * BINW + sid * zslice, zslice)])
        plsc.subcore_barrier()
        return 0
    lax.fori_loop(0, NBINS // NCORES, _round, 0)


@jax.jit
def kernel(inputs_values, unpool_mat):
    vals_bits = lax.bitcast_convert_type(inputs_values.reshape(-1),
                                         jnp.int32)
    idx = unpool_mat.reshape(-1)
    mesh = plsc.VectorSubcoreMesh(core_axis_name="c", subcore_axis_name="s")
    params = pltpu.CompilerParams(needs_layout_passes=False)

    stgi, stgv, pref = pl.kernel(
        _route_body,
        out_type=(jax.ShapeDtypeStruct((NSTG,), jnp.int32),
                  jax.ShapeDtypeStruct((NSTG,), jnp.int32),
                  jax.ShapeDtypeStruct(((NB_PAD + 1) * NSTRIP,), jnp.int32)),
        mesh=mesh,
        scratch_types=[
            pltpu.VMEM((WSZ,), jnp.int32),
            pltpu.VMEM((WSZ,), jnp.int32),
            pltpu.VMEM((WSZ,), jnp.int32),
            pltpu.VMEM((16 * NB_PAD,), jnp.int32),
            pltpu.VMEM((NB_PAD,), jnp.int32),
            pltpu.VMEM((NB_PAD,), jnp.int32),
            pltpu.VMEM((NB_PAD,), jnp.int32),
        ],
        compiler_params=params,
    )(idx, vals_bits)

    stgv_f = lax.bitcast_convert_type(stgv, jnp.float32)
    out = pl.kernel(
        _acc_body,
        out_type=jax.ShapeDtypeStruct((N_OUT,), jnp.float32),
        mesh=mesh,
        scratch_types=[
            pltpu.VMEM((RW,), jnp.int32),
            pltpu.VMEM((RW,), jnp.float32),
            pltpu.VMEM((2 * NSTRIP,), jnp.int32),
            pltpu.VMEM((ZB,), jnp.float32),
            pltpu.SMEM((128,), jnp.int32),
            pltpu.VMEM_SHARED((BINW + TRASH,), jnp.float32),
            pltpu.VMEM_SHARED((NSUB * 128,), jnp.int32),
        ],
        compiler_params=params,
    )(stgi, stgv_f, pref)

    return out.reshape(inputs_values.shape[0], 384, 384, 96)


# speedup vs baseline: 14.7560x; 1.1204x over previous
import jax
import jax.numpy as jnp
from jax import lax
from jax.experimental import pallas as pl
from jax.experimental.pallas import tpu as pltpu
from jax.experimental.pallas import tpu_sc as plsc

N_OUT = 4 * 384 * 384 * 96
N_UPD = 4 * 192 * 192 * 96
SHIFT = 20
BINW = 1 << SHIFT
NBINS = N_OUT // BINW
NB_PAD = 64
NCORES = 2
NSUB = 16
NW = NCORES * NSUB
CHUNK = N_UPD // NW
WSZ = 36864
WPT = CHUNK // WSZ
NSTRIP = NW * WPT
RW = 1024
NSTG = N_UPD + RW
TRASH = 16 * NSUB
SPT = NSTRIP // NSUB
ZB = 8192


def _route_body(idx_hbm, val_hbm, stgi_hbm, stgv_hbm, pref_hbm,
                a_v, dstb_v, c_v, whist_v, wctr_v, pref_v, dstp_v):
    wid = lax.axis_index("s") * NCORES + lax.axis_index("c")
    base = wid * CHUNK
    lanes = lax.iota(jnp.int32, 16)
    lanebase = lanes * NB_PAD
    ones = jnp.ones((16,), jnp.int32)
    mask20 = jnp.int32(BINW - 1)

    def _win(w, _):
        col = wid * WPT + w
        pltpu.sync_copy(idx_hbm.at[pl.ds(base + w * WSZ, WSZ)], a_v)

        def _z(i, _):
            whist_v[pl.ds(i * 16, 16)] = jnp.zeros((16,), jnp.int32)
            return 0
        lax.fori_loop(0, (16 * NB_PAD) // 16, _z, 0)

        def _h(j, _):
            iv = a_v[pl.ds(j * 16, 16)]
            pos = lax.shift_right_logical(iv, SHIFT) + lanebase
            plsc.addupdate_scatter(whist_v, [pos], ones)
            return 0
        lax.fori_loop(0, WSZ // 16, _h, 0)

        carry = jnp.int32(0)
        for g in range(NB_PAD // 16):
            tot = jnp.zeros((16,), jnp.int32)
            for l in range(16):
                tot = tot + whist_v[pl.ds(l * NB_PAD + g * 16, 16)]
            excl = plsc.cumsum(tot) - tot + carry
            wctr_v[pl.ds(g * 16, 16)] = excl
            pref_v[pl.ds(g * 16, 16)] = excl
            dstp_v[pl.ds(g * 16, 16)] = (g * 16 + lanes) * NSTRIP + col
            carry = carry + jnp.sum(tot)

        pltpu.sync_copy(pref_v, pref_hbm.at[dstp_v])

        def _r(j, _):
            iv = a_v[pl.ds(j * 16, 16)]
            b = lax.shift_right_logical(iv, SHIFT)
            cnt, last = plsc.scan_count(b)
            bse = plsc.load_gather(wctr_v, [b])
            dst = bse + cnt - 1
            dstb_v[pl.ds(j * 16, 16)] = dst
            plsc.store_scatter(c_v, [dst], iv & mask20)
            plsc.addupdate_scatter(wctr_v, [b], cnt, mask=last)
            return 0
        lax.fori_loop(0, WSZ // 16, _r, 0)
        pltpu.sync_copy(c_v, stgi_hbm.at[pl.ds(col * WSZ, WSZ)])

        pltpu.sync_copy(val_hbm.at[pl.ds(base + w * WSZ, WSZ)], a_v)

        def _r2(j, _):
            vv = a_v[pl.ds(j * 16, 16)]
            dst = dstb_v[pl.ds(j * 16, 16)]
            plsc.store_scatter(c_v, [dst], vv)
            return 0
        lax.fori_loop(0, WSZ // 16, _r2, 0)
        pltpu.sync_copy(c_v, stgv_hbm.at[pl.ds(col * WSZ, WSZ)])
        return 0
    lax.fori_loop(0, WPT, _win, 0)


def _acc_body(stgi_hbm, stgv_hbm, pref_hbm, out_hbm,
              ridx_v, rval_v, prow_v, zero_v, pref_sm, sem1, sem2, acc,
              pref_sp):
    cid = lax.axis_index("c")
    sid = lax.axis_index("s")
    lanes = lax.iota(jnp.int32, 16)
    trash_vec = jnp.int32(BINW) + sid * 16 + lanes
    zslice = BINW // NSUB

    def _zz(i, _):
        zero_v[pl.ds(i * 16, 16)] = jnp.zeros((16,), jnp.float32)
        return 0
    lax.fori_loop(0, ZB // 16, _zz, 0)

    def _round(r, _):
        b = NCORES * r + cid

        zds = [pltpu.async_copy(
                   zero_v, acc.at[pl.ds(sid * zslice + z * ZB, ZB)], sem1)
               for z in range(zslice // ZB)]
        for d in zds:
            d.wait()

        pltpu.sync_copy(pref_hbm.at[pl.ds(b * NSTRIP, NSTRIP)],
                        prow_v.at[pl.ds(0, NSTRIP)])
        pltpu.sync_copy(pref_hbm.at[pl.ds((b + 1) * NSTRIP, NSTRIP)],
                        prow_v.at[pl.ds(NSTRIP, NSTRIP)])
        al_s = pl.multiple_of((sid * SPT) & ~jnp.int32(7), 8)
        pltpu.sync_copy(prow_v.at[pl.ds(al_s, 64)],
                        pref_sp.at[pl.ds(sid * 128, 64)])
        pltpu.sync_copy(prow_v.at[pl.ds(NSTRIP + al_s, 64)],
                        pref_sp.at[pl.ds(sid * 128 + 64, 64)])
        pltpu.sync_copy(pref_sp.at[pl.ds(sid * 128, 128)], pref_sm)
        plsc.subcore_barrier()

        def _run(k, _):
            s_k = sid * SPT + k
            st = s_k * WSZ + pref_sm[s_k - al_s]
            en = s_k * WSZ + pref_sm[64 + s_k - al_s]
            al = pl.multiple_of(st & ~jnp.int32(7), 8)
            nwin = jnp.where(
                en > st,
                lax.shift_right_logical(en - al + RW - 1, 10), 0)

            def _w(w, _):
                wb = al + w * RW
                d1 = pltpu.async_copy(stgi_hbm.at[pl.ds(wb, RW)],
                                      ridx_v, sem1)
                d2 = pltpu.async_copy(stgv_hbm.at[pl.ds(wb, RW)],
                                      rval_v, sem2)
                d1.wait()
                d2.wait()

                def _v(j, _):
                    pos = wb + j * 16 + lanes
                    iv = ridx_v[pl.ds(j * 16, 16)]
                    ok = (pos >= st) & (pos < en)
                    ridx_v[pl.ds(j * 16, 16)] = jnp.where(ok, iv, trash_vec)
                    return 0
                lax.fori_loop(0, RW // 16, _v, 0)

                pltpu.sync_copy(rval_v, acc.at[ridx_v], add=True)
                return 0
            lax.fori_loop(0, nwin, _w, 0)
            return 0
        lax.fori_loop(0, SPT, _run, 0)
        plsc.subcore_barrier()

        pltpu.sync_copy(acc.at[pl.ds(sid * zslice, zslice)],
                        out_hbm.at[pl.ds(b * BINW + sid * zslice, zslice)])
        plsc.subcore_barrier()
        return 0
    lax.fori_loop(0, NBINS // NCORES, _round, 0)


@jax.jit
def kernel(inputs_values, unpool_mat):
    vals_bits = lax.bitcast_convert_type(inputs_values.reshape(-1),
                                         jnp.int32)
    idx = unpool_mat.reshape(-1)
    mesh = plsc.VectorSubcoreMesh(core_axis_name="c", subcore_axis_name="s")
    params = pltpu.CompilerParams(needs_layout_passes=False)

    stgi, stgv, pref = pl.kernel(
        _route_body,
        out_type=(jax.ShapeDtypeStruct((NSTG,), jnp.int32),
                  jax.ShapeDtypeStruct((NSTG,), jnp.int32),
                  jax.ShapeDtypeStruct(((NB_PAD + 1) * NSTRIP,), jnp.int32)),
        mesh=mesh,
        scratch_types=[
            pltpu.VMEM((WSZ,), jnp.int32),
            pltpu.VMEM((WSZ,), jnp.int32),
            pltpu.VMEM((WSZ,), jnp.int32),
            pltpu.VMEM((16 * NB_PAD,), jnp.int32),
            pltpu.VMEM((NB_PAD,), jnp.int32),
            pltpu.VMEM((NB_PAD,), jnp.int32),
            pltpu.VMEM((NB_PAD,), jnp.int32),
        ],
        compiler_params=params,
    )(idx, vals_bits)

    stgv_f = lax.bitcast_convert_type(stgv, jnp.float32)
    out = pl.kernel(
        _acc_body,
        out_type=jax.ShapeDtypeStruct((N_OUT,), jnp.float32),
        mesh=mesh,
        scratch_types=[
            pltpu.VMEM((RW,), jnp.int32),
            pltpu.VMEM((RW,), jnp.float32),
            pltpu.VMEM((2 * NSTRIP,), jnp.int32),
            pltpu.VMEM((ZB,), jnp.float32),
            pltpu.SMEM((128,), jnp.int32),
            pltpu.SemaphoreType.DMA,
            pltpu.SemaphoreType.DMA,
            pltpu.VMEM_SHARED((BINW + TRASH,), jnp.float32),
            pltpu.VMEM_SHARED((NSUB * 128,), jnp.int32),
        ],
        compiler_params=params,
    )(stgi, stgv_f, pref)

    return out.reshape(inputs_values.shape[0], 384, 384, 96)
